# baseline (speedup 1.0000x reference)
import jax
import jax.numpy as jnp
from jax.experimental import pallas as pl
from jax.experimental.pallas import tpu as pltpu

D_MODEL = 1024
NUM_EXPERTS = 8
N_TOKENS = 4096
TOKEN_BLOCK = 2048


def _moe_body(x_ref, gw_ref, gb_ref, ew_ref, eb_ref, out_ref, cm_ref):
    e = pl.program_id(1)

    @pl.when(e == 0)
    def _gating():
        logits = (
            jnp.dot(x_ref[...], gw_ref[...], preferred_element_type=jnp.float32)
            + gb_ref[...]
        )
        iota = jax.lax.broadcasted_iota(jnp.int32, logits.shape, 1)
        m1 = jnp.max(logits, axis=-1, keepdims=True)
        idx1 = jnp.min(
            jnp.where(logits == m1, iota, NUM_EXPERTS), axis=-1, keepdims=True
        )
        one1 = iota == idx1
        masked = jnp.where(one1, -jnp.inf, logits)
        m2 = jnp.max(masked, axis=-1, keepdims=True)
        idx2 = jnp.min(
            jnp.where(masked == m2, iota, NUM_EXPERTS), axis=-1, keepdims=True
        )
        one2 = iota == idx2
        c1 = 1.0 / (1.0 + jnp.exp(m2 - m1))
        c2 = 1.0 - c1
        cm_ref[...] = jnp.where(one1, c1, 0.0) + jnp.where(one2, c2, 0.0)

    iota = jax.lax.broadcasted_iota(jnp.int32, (TOKEN_BLOCK, NUM_EXPERTS), 1)
    csel = jnp.sum(
        jnp.where(iota == e, cm_ref[...], 0.0), axis=-1, keepdims=True
    )
    xs = x_ref[...] * csel
    w = ew_ref[0]
    h = D_MODEL // 2
    y0 = jnp.dot(xs, w[:, :h], preferred_element_type=jnp.float32)

    @pl.when(e == 0)
    def _init0():
        out_ref[:, :h] = y0 + jnp.dot(
            cm_ref[...], eb_ref[:, :h], preferred_element_type=jnp.float32
        )

    @pl.when(e != 0)
    def _acc0():
        out_ref[:, :h] = out_ref[:, :h] + y0

    y1 = jnp.dot(xs, w[:, h:], preferred_element_type=jnp.float32)

    @pl.when(e == 0)
    def _init1():
        out_ref[:, h:] = y1 + jnp.dot(
            cm_ref[...], eb_ref[:, h:], preferred_element_type=jnp.float32
        )

    @pl.when(e != 0)
    def _acc1():
        out_ref[:, h:] = out_ref[:, h:] + y1


@jax.jit
def kernel(x, gate_W, gate_b, expert_W, expert_b):
    n_tb = N_TOKENS // TOKEN_BLOCK
    gb2 = gate_b.reshape(1, NUM_EXPERTS)
    return pl.pallas_call(
        _moe_body,
        grid=(n_tb, NUM_EXPERTS),
        in_specs=[
            pl.BlockSpec((TOKEN_BLOCK, D_MODEL), lambda t, e: (t, 0)),
            pl.BlockSpec((D_MODEL, NUM_EXPERTS), lambda t, e: (0, 0)),
            pl.BlockSpec((1, NUM_EXPERTS), lambda t, e: (0, 0)),
            pl.BlockSpec((1, D_MODEL, D_MODEL), lambda t, e: (e, 0, 0)),
            pl.BlockSpec((NUM_EXPERTS, D_MODEL), lambda t, e: (0, 0)),
        ],
        out_specs=pl.BlockSpec((TOKEN_BLOCK, D_MODEL), lambda t, e: (t, 0)),
        out_shape=jax.ShapeDtypeStruct((N_TOKENS, D_MODEL), jnp.float32),
        scratch_shapes=[
            pltpu.VMEM((TOKEN_BLOCK, NUM_EXPERTS), jnp.float32),
        ],
    )(x, gate_W, gb2, expert_W, expert_b)


# speedup vs baseline: 1.0392x; 1.0392x over previous
import jax
import jax.numpy as jnp
from jax.experimental import pallas as pl
from jax.experimental.pallas import tpu as pltpu

D_MODEL = 1024
NUM_EXPERTS = 8
N_TOKENS = 4096
TOKEN_BLOCK = 2048


def _moe_body(x_ref, gw_ref, gb_ref, ew_ref, eb_ref, out_ref, cm_ref):
    e = pl.program_id(1)

    @pl.when(e == 0)
    def _gating():
        logits = (
            jnp.dot(x_ref[...], gw_ref[...], preferred_element_type=jnp.float32)
            + gb_ref[...]
        )
        iota = jax.lax.broadcasted_iota(jnp.int32, logits.shape, 1)
        m1 = jnp.max(logits, axis=-1, keepdims=True)
        idx1 = jnp.min(
            jnp.where(logits == m1, iota, NUM_EXPERTS), axis=-1, keepdims=True
        )
        one1 = iota == idx1
        masked = jnp.where(one1, -jnp.inf, logits)
        m2 = jnp.max(masked, axis=-1, keepdims=True)
        idx2 = jnp.min(
            jnp.where(masked == m2, iota, NUM_EXPERTS), axis=-1, keepdims=True
        )
        one2 = iota == idx2
        c1 = 1.0 / (1.0 + jnp.exp(m2 - m1))
        c2 = 1.0 - c1
        cm_ref[...] = jnp.where(one1, c1, 0.0) + jnp.where(one2, c2, 0.0)

    iota = jax.lax.broadcasted_iota(jnp.int32, (TOKEN_BLOCK, NUM_EXPERTS), 1)
    csel = jnp.sum(
        jnp.where(iota == e, cm_ref[...], 0.0), axis=-1, keepdims=True
    )
    y = jnp.dot(x_ref[...], ew_ref[0], preferred_element_type=jnp.float32)

    @pl.when(e == 0)
    def _init():
        out_ref[...] = csel * y + jnp.dot(
            cm_ref[...], eb_ref[...], preferred_element_type=jnp.float32
        )

    @pl.when(e != 0)
    def _acc():
        out_ref[...] = out_ref[...] + csel * y


@jax.jit
def kernel(x, gate_W, gate_b, expert_W, expert_b):
    n_tb = N_TOKENS // TOKEN_BLOCK
    gb2 = gate_b.reshape(1, NUM_EXPERTS)
    return pl.pallas_call(
        _moe_body,
        grid=(n_tb, NUM_EXPERTS),
        in_specs=[
            pl.BlockSpec((TOKEN_BLOCK, D_MODEL), lambda t, e: (t, 0)),
            pl.BlockSpec((D_MODEL, NUM_EXPERTS), lambda t, e: (0, 0)),
            pl.BlockSpec((1, NUM_EXPERTS), lambda t, e: (0, 0)),
            pl.BlockSpec((1, D_MODEL, D_MODEL), lambda t, e: (e, 0, 0)),
            pl.BlockSpec((NUM_EXPERTS, D_MODEL), lambda t, e: (0, 0)),
        ],
        out_specs=pl.BlockSpec((TOKEN_BLOCK, D_MODEL), lambda t, e: (t, 0)),
        out_shape=jax.ShapeDtypeStruct((N_TOKENS, D_MODEL), jnp.float32),
        scratch_shapes=[
            pltpu.VMEM((TOKEN_BLOCK, NUM_EXPERTS), jnp.float32),
        ],
    )(x, gate_W, gb2, expert_W, expert_b)


# speedup vs baseline: 1.1241x; 1.0817x over previous
import jax
import jax.numpy as jnp
from jax.experimental import pallas as pl
from jax.experimental.pallas import tpu as pltpu

D_MODEL = 1024
NUM_EXPERTS = 8
N_TOKENS = 4096
TOKEN_BLOCK = 1024


def _moe_body(x_ref, gw_ref, gb_ref, ew_ref, eb_ref, out_ref, cm_ref):
    e = pl.program_id(1)

    @pl.when(e == 0)
    def _gating():
        logits = (
            jnp.dot(x_ref[...], gw_ref[...], preferred_element_type=jnp.float32)
            + gb_ref[...]
        )
        iota = jax.lax.broadcasted_iota(jnp.int32, logits.shape, 1)
        m1 = jnp.max(logits, axis=-1, keepdims=True)
        idx1 = jnp.min(
            jnp.where(logits == m1, iota, NUM_EXPERTS), axis=-1, keepdims=True
        )
        one1 = iota == idx1
        masked = jnp.where(one1, -jnp.inf, logits)
        m2 = jnp.max(masked, axis=-1, keepdims=True)
        idx2 = jnp.min(
            jnp.where(masked == m2, iota, NUM_EXPERTS), axis=-1, keepdims=True
        )
        one2 = iota == idx2
        c1 = 1.0 / (1.0 + jnp.exp(m2 - m1))
        c2 = 1.0 - c1
        cm_ref[...] = jnp.where(one1, c1, 0.0) + jnp.where(one2, c2, 0.0)

    iota = jax.lax.broadcasted_iota(jnp.int32, (TOKEN_BLOCK, NUM_EXPERTS), 1)
    ca = jnp.sum(
        jnp.where(iota == 2 * e, cm_ref[...], 0.0), axis=-1, keepdims=True
    )
    cb = jnp.sum(
        jnp.where(iota == 2 * e + 1, cm_ref[...], 0.0), axis=-1, keepdims=True
    )
    ya = jnp.dot(x_ref[...], ew_ref[0], preferred_element_type=jnp.float32)
    yb = jnp.dot(x_ref[...], ew_ref[1], preferred_element_type=jnp.float32)
    y = ca * ya + cb * yb

    @pl.when(e == 0)
    def _init():
        out_ref[...] = y + jnp.dot(
            cm_ref[...], eb_ref[...], preferred_element_type=jnp.float32
        )

    @pl.when(e != 0)
    def _acc():
        out_ref[...] = out_ref[...] + y


@jax.jit
def kernel(x, gate_W, gate_b, expert_W, expert_b):
    n_tb = N_TOKENS // TOKEN_BLOCK
    gb2 = gate_b.reshape(1, NUM_EXPERTS)
    return pl.pallas_call(
        _moe_body,
        grid=(n_tb, NUM_EXPERTS // 2),
        in_specs=[
            pl.BlockSpec((TOKEN_BLOCK, D_MODEL), lambda t, e: (t, 0)),
            pl.BlockSpec((D_MODEL, NUM_EXPERTS), lambda t, e: (0, 0)),
            pl.BlockSpec((1, NUM_EXPERTS), lambda t, e: (0, 0)),
            pl.BlockSpec((2, D_MODEL, D_MODEL), lambda t, e: (e, 0, 0)),
            pl.BlockSpec((NUM_EXPERTS, D_MODEL), lambda t, e: (0, 0)),
        ],
        out_specs=pl.BlockSpec((TOKEN_BLOCK, D_MODEL), lambda t, e: (t, 0)),
        out_shape=jax.ShapeDtypeStruct((N_TOKENS, D_MODEL), jnp.float32),
        scratch_shapes=[
            pltpu.VMEM((TOKEN_BLOCK, NUM_EXPERTS), jnp.float32),
        ],
    )(x, gate_W, gb2, expert_W, expert_b)
